# baseline (speedup 1.0000x reference)
import jax
import jax.numpy as jnp
from jax.experimental import pallas as pl
from jax.experimental.pallas import tpu as pltpu

_VMEM_LIMIT_BYTES = 48 * 1024 * 1024


def _mlp_kernel(x_ref, w0_ref, w1_ref, w2_ref, w3_ref,
                b0_ref, b1_ref, b2_ref, b3_ref, o_ref):
    tm = x_ref.shape[0]
    half = tm // 2
    hs = [x_ref[:half].astype(jnp.bfloat16), x_ref[half:].astype(jnp.bfloat16)]
    for w_ref, b_ref, last in (
        (w0_ref, b0_ref, False),
        (w1_ref, b1_ref, False),
        (w2_ref, b2_ref, False),
        (w3_ref, b3_ref, True),
    ):
        w = w_ref[...]
        b = b_ref[...]
        accs = [jnp.dot(h, w, preferred_element_type=jnp.float32) for h in hs]
        if last:
            o_ref[:half] = jnp.maximum(accs[0] + b, 0.0)
            o_ref[half:] = jnp.maximum(accs[1] + b, 0.0)
        else:
            bh = b.astype(jnp.bfloat16)
            hs = [jnp.maximum(a.astype(jnp.bfloat16) + bh, jnp.bfloat16(0.0))
                  for a in accs]


def _fused_mlp(h, ws, bs, tm):
    M, F = h.shape
    grid = (M // tm,)
    row_spec = pl.BlockSpec((tm, F), lambda i: (i, 0))
    w_spec = pl.BlockSpec((F, F), lambda i: (0, 0))
    b_spec = pl.BlockSpec((1, F), lambda i: (0, 0))
    return pl.pallas_call(
        _mlp_kernel,
        out_shape=jax.ShapeDtypeStruct((M, F), jnp.float32),
        grid=grid,
        in_specs=[row_spec] + [w_spec] * 4 + [b_spec] * 4,
        out_specs=row_spec,
        compiler_params=pltpu.CompilerParams(
            dimension_semantics=("arbitrary",),
            vmem_limit_bytes=_VMEM_LIMIT_BYTES,
        ),
        cost_estimate=pl.CostEstimate(
            flops=2 * M * F * F * 4,
            transcendentals=0,
            bytes_accessed=4 * (M * F + F + M * F) + 2 * 4 * F * F,
        ),
    )(h, *ws, *bs)


def kernel(x, w0, b0, w1, b1, w2, b2, w3, b3):
    bcz, seq_len, in_f = x.shape
    h = x.reshape(-1, in_f)
    M = h.shape[0]
    tm = 1024 if M % 1024 == 0 else (512 if M % 512 == 0 else 256)
    ws = [w.astype(jnp.bfloat16) for w in (w0, w1, w2, w3)]
    bs = [b.reshape(1, -1) for b in (b0, b1, b2, b3)]
    out = _fused_mlp(h, ws, bs, tm)
    return out.reshape(bcz, seq_len, -1)


# speedup vs baseline: 1.0521x; 1.0521x over previous
import jax
import jax.numpy as jnp
from jax.experimental import pallas as pl
from jax.experimental.pallas import tpu as pltpu

_VMEM_LIMIT_BYTES = 48 * 1024 * 1024


def _mlp_kernel(x_ref, w0_ref, w1_ref, w2_ref, w3_ref,
                b0_ref, b1_ref, b2_ref, b3_ref, o_ref):
    tm = x_ref.shape[0]
    half = tm // 2
    nh = o_ref.shape[1] // 2
    hs = [x_ref[:half], x_ref[half:]]
    for w_ref, b_ref, last in (
        (w0_ref, b0_ref, False),
        (w1_ref, b1_ref, False),
        (w2_ref, b2_ref, False),
        (w3_ref, b3_ref, True),
    ):
        b = b_ref[...]
        bh = b.astype(jnp.bfloat16)
        nxt = []
        for ci, h in enumerate(hs):
            accl = jnp.dot(h, w_ref[:, :nh], preferred_element_type=jnp.float32)
            accr = jnp.dot(h, w_ref[:, nh:], preferred_element_type=jnp.float32)
            if last:
                sl = slice(0, half) if ci == 0 else slice(half, tm)
                o_ref[sl, :nh] = jnp.maximum(accl + b[:, :nh], 0.0)
                o_ref[sl, nh:] = jnp.maximum(accr + b[:, nh:], 0.0)
            else:
                al = jnp.maximum(accl + b[:, :nh], 0.0)
                ar = jnp.maximum(accr + b[:, nh:], 0.0)
                nxt.append(jnp.concatenate([al, ar], axis=1))
        hs = nxt


def _fused_mlp(h, ws, bs, tm):
    M, F = h.shape
    grid = (M // tm,)
    row_spec = pl.BlockSpec((tm, F), lambda i: (i, 0))
    w_spec = pl.BlockSpec((F, F), lambda i: (0, 0))
    b_spec = pl.BlockSpec((1, F), lambda i: (0, 0))
    return pl.pallas_call(
        _mlp_kernel,
        out_shape=jax.ShapeDtypeStruct((M, F), jnp.float32),
        grid=grid,
        in_specs=[row_spec] + [w_spec] * 4 + [b_spec] * 4,
        out_specs=row_spec,
        compiler_params=pltpu.CompilerParams(
            dimension_semantics=("arbitrary",),
            vmem_limit_bytes=_VMEM_LIMIT_BYTES,
        ),
        cost_estimate=pl.CostEstimate(
            flops=2 * M * F * F * 4,
            transcendentals=0,
            bytes_accessed=4 * (M * F + F + M * F) + 2 * 4 * F * F,
        ),
    )(h, *ws, *bs)


def kernel(x, w0, b0, w1, b1, w2, b2, w3, b3):
    bcz, seq_len, in_f = x.shape
    h = x.reshape(-1, in_f)
    M = h.shape[0]
    tm = 1024 if M % 1024 == 0 else (512 if M % 512 == 0 else 256)
    ws = [w0, w1, w2, w3]
    bs = [b.reshape(1, -1) for b in (b0, b1, b2, b3)]
    out = _fused_mlp(h, ws, bs, tm)
    return out.reshape(bcz, seq_len, -1)
